# baseline (speedup 1.0000x reference)
import functools

import jax
import jax.numpy as jnp
from jax import lax
from jax.experimental import pallas as pl
from jax.experimental.pallas import tpu as pltpu
from jax.experimental.pallas import tpu_sc as plsc

GR = 256
LANES = 16
C = 1024
NC = 2
NS = 16
NW = NC * NS


def _axis_terms(p):
    p = jnp.minimum(jnp.maximum(p, 0.0), 1.0 - 1e-6)
    gc = p * float(GR)
    gc = jnp.minimum(jnp.maximum(gc, 0.0), float(GR - 1))
    i0 = gc.astype(jnp.int32)
    i1 = jnp.minimum(i0 + 1, GR - 1)
    d = gc - i0.astype(jnp.float32)
    return i0, i1, d


def _make_sc_interp(npad, niter):
    mesh = plsc.VectorSubcoreMesh(core_axis_name="c", subcore_axis_name="s")

    @functools.partial(
        pl.kernel,
        mesh=mesh,
        compiler_params=pltpu.CompilerParams(needs_layout_passes=False),
        out_type=jax.ShapeDtypeStruct((npad,), jnp.float32),
        scratch_types=[
            pltpu.VMEM((3 * C,), jnp.float32),
            pltpu.VMEM((3 * C,), jnp.float32),
            pltpu.VMEM((8 * C,), jnp.int32),
            pltpu.VMEM((8 * C,), jnp.int32),
            pltpu.VMEM((3 * C,), jnp.float32),
            pltpu.VMEM((3 * C,), jnp.float32),
            pltpu.VMEM((8 * C,), jnp.float32),
            pltpu.VMEM((8 * C,), jnp.float32),
            pltpu.VMEM((C,), jnp.float32),
            pltpu.SemaphoreType.DMA,
            pltpu.SemaphoreType.DMA,
            pltpu.SemaphoreType.DMA,
            pltpu.SemaphoreType.DMA,
        ],
    )
    def sc_interp(xyz_hbm, gridf_hbm, out_hbm, pts_v0, pts_v1, idx_v0,
                  idx_v1, wt_v0, wt_v1, val_v0, val_v1, out_v, sem_p0,
                  sem_p1, sem_g0, sem_g1):
        pts_v = (pts_v0, pts_v1)
        idx_v = (idx_v0, idx_v1)
        wt_v = (wt_v0, wt_v1)
        val_v = (val_v0, val_v1)
        sem_p = (sem_p0, sem_p1)
        sem_g = (sem_g0, sem_g1)
        wid = lax.axis_index("s") * NC + lax.axis_index("c")

        def chunk_base(t):
            return jnp.minimum((wid + NW * t) * C, npad - C)

        def pts_copy(t, b):
            base = chunk_base(t)
            return pltpu.make_async_copy(
                xyz_hbm.at[pl.ds(base * 3, 3 * C)], pts_v[b], sem_p[b])

        def gather_copy(k, b):
            return pltpu.make_async_copy(
                gridf_hbm.at[idx_v[b].at[pl.ds(k * C, C)]],
                val_v[b].at[pl.ds(k * C, C)], sem_g[b])

        def compute_idx(b):
            iv, wv, pv = idx_v[b], wt_v[b], pts_v[b]

            def vec_body(j, carry2):
                sb = j * LANES
                flat = (sb + lax.iota(jnp.int32, LANES)) * 3
                px = plsc.load_gather(pv, [flat])
                py = plsc.load_gather(pv, [flat + 1])
                pz = plsc.load_gather(pv, [flat + 2])
                x0, x1, xd = _axis_terms(px)
                y0, y1, yd = _axis_terms(py)
                z0, z1, zd = _axis_terms(pz)
                x0s = x0 << 16
                x1s = x1 << 16
                y0s = y0 << 8
                y1s = y1 << 8
                b00 = x0s + y0s
                b01 = x0s + y1s
                b10 = x1s + y0s
                b11 = x1s + y1s
                iv[pl.ds(0 * C + sb, LANES)] = b00 + z0
                iv[pl.ds(1 * C + sb, LANES)] = b00 + z1
                iv[pl.ds(2 * C + sb, LANES)] = b01 + z0
                iv[pl.ds(3 * C + sb, LANES)] = b01 + z1
                iv[pl.ds(4 * C + sb, LANES)] = b10 + z0
                iv[pl.ds(5 * C + sb, LANES)] = b10 + z1
                iv[pl.ds(6 * C + sb, LANES)] = b11 + z0
                iv[pl.ds(7 * C + sb, LANES)] = b11 + z1
                wv[pl.ds(0 * C + sb, LANES)] = xd
                wv[pl.ds(1 * C + sb, LANES)] = yd
                wv[pl.ds(2 * C + sb, LANES)] = zd
                return carry2

            lax.fori_loop(0, C // LANES, vec_body, 0)

        def combine_out(t, b):
            vv, wv = val_v[b], wt_v[b]
            for k in range(8):
                gather_copy(k, b).wait()

            def mix_body(j, carry2):
                sb = j * LANES
                v000 = vv[pl.ds(0 * C + sb, LANES)]
                v001 = vv[pl.ds(1 * C + sb, LANES)]
                v010 = vv[pl.ds(2 * C + sb, LANES)]
                v011 = vv[pl.ds(3 * C + sb, LANES)]
                v100 = vv[pl.ds(4 * C + sb, LANES)]
                v101 = vv[pl.ds(5 * C + sb, LANES)]
                v110 = vv[pl.ds(6 * C + sb, LANES)]
                v111 = vv[pl.ds(7 * C + sb, LANES)]
                xd = wv[pl.ds(0 * C + sb, LANES)]
                yd = wv[pl.ds(1 * C + sb, LANES)]
                zd = wv[pl.ds(2 * C + sb, LANES)]
                c00 = v000 + (v100 - v000) * xd
                c01 = v001 + (v101 - v001) * xd
                c10 = v010 + (v110 - v010) * xd
                c11 = v011 + (v111 - v011) * xd
                c0 = c00 + (c10 - c00) * yd
                c1 = c01 + (c11 - c01) * yd
                out_v[pl.ds(sb, LANES)] = c0 + (c1 - c0) * zd
                return carry2

            lax.fori_loop(0, C // LANES, mix_body, 0)
            pltpu.sync_copy(out_v, out_hbm.at[pl.ds(chunk_base(t), C)])

        def half_iter(t, cur):
            nxt = 1 - cur
            pts_copy(t, cur).wait()
            compute_idx(cur)
            for k in range(8):
                gather_copy(k, cur).start()

            @pl.when(t + 1 < niter)
            def _():
                pts_copy(t + 1, nxt).start()

            @pl.when(t >= 1)
            def _():
                combine_out(t - 1, nxt)

        pts_copy(0, 0).start()

        def pair_body(tt, carry):
            half_iter(2 * tt, 0)
            half_iter(2 * tt + 1, 1)
            return carry

        lax.fori_loop(0, niter // 2, pair_body, 0)
        if niter % 2:
            half_iter(niter - 1, 0)
        last = niter - 1
        combine_out(last, last % 2)

    return sc_interp


def kernel(points, grid):
    npts = points.shape[0]
    niter = -(-npts // (C * NW))
    xyz = points.reshape(-1)
    gridf = grid.reshape(-1)
    return _make_sc_interp(npts, niter)(xyz, gridf)


# speedup vs baseline: 9.1011x; 9.1011x over previous
import functools

import jax
import jax.numpy as jnp
from jax import lax
from jax.experimental import pallas as pl
from jax.experimental.pallas import tpu as pltpu
from jax.experimental.pallas import tpu_sc as plsc

GR = 256
LANES = 16
C = 1024
NC = 2
NS = 16
NW = NC * NS


def _axis_terms(p):
    p = jnp.minimum(jnp.maximum(p, 0.0), 1.0 - 1e-6)
    gc = p * float(GR)
    gc = jnp.minimum(jnp.maximum(gc, 0.0), float(GR - 1))
    i0 = gc.astype(jnp.int32)
    i1 = jnp.minimum(i0 + 1, GR - 1)
    d = gc - i0.astype(jnp.float32)
    return i0, i1, d


def _make_sc_interp(npad, niter):
    mesh = plsc.VectorSubcoreMesh(core_axis_name="c", subcore_axis_name="s")

    @functools.partial(
        pl.kernel,
        mesh=mesh,
        compiler_params=pltpu.CompilerParams(needs_layout_passes=False),
        out_type=jax.ShapeDtypeStruct((npad,), jnp.float32),
        scratch_types=[
            pltpu.VMEM((3 * C,), jnp.float32),
            pltpu.VMEM((3 * C,), jnp.float32),
            pltpu.VMEM((8 * C,), jnp.int32),
            pltpu.VMEM((8 * C,), jnp.int32),
            pltpu.VMEM((3 * C,), jnp.float32),
            pltpu.VMEM((3 * C,), jnp.float32),
            pltpu.VMEM((8 * C,), jnp.float32),
            pltpu.VMEM((8 * C,), jnp.float32),
            pltpu.VMEM((C,), jnp.float32),
            pltpu.SemaphoreType.DMA,
            pltpu.SemaphoreType.DMA,
            pltpu.SemaphoreType.DMA,
            pltpu.SemaphoreType.DMA,
        ],
    )
    def sc_interp(xs_hbm, ys_hbm, zs_hbm, gridf_hbm, out_hbm, pts_v0, pts_v1, idx_v0,
                  idx_v1, wt_v0, wt_v1, val_v0, val_v1, out_v, sem_p0,
                  sem_p1, sem_g0, sem_g1):
        pts_v = (pts_v0, pts_v1)
        idx_v = (idx_v0, idx_v1)
        wt_v = (wt_v0, wt_v1)
        val_v = (val_v0, val_v1)
        sem_p = (sem_p0, sem_p1)
        sem_g = (sem_g0, sem_g1)
        wid = lax.axis_index("s") * NC + lax.axis_index("c")

        def chunk_base(t):
            return jnp.minimum((wid + NW * t) * C, npad - C)

        def pts_copy(t, b):
            base = chunk_base(t)
            return [
                pltpu.make_async_copy(src.at[pl.ds(base, C)],
                                      pts_v[b].at[pl.ds(i * C, C)], sem_p[b])
                for i, src in enumerate((xs_hbm, ys_hbm, zs_hbm))
            ]

        def pts_start(t, b):
            for cp in pts_copy(t, b):
                cp.start()

        def pts_wait(t, b):
            for cp in pts_copy(t, b):
                cp.wait()

        def gather_copy(k, b):
            return pltpu.make_async_copy(
                gridf_hbm.at[idx_v[b].at[pl.ds(k * C, C)]],
                val_v[b].at[pl.ds(k * C, C)], sem_g[b])

        def compute_idx(b):
            iv, wv, pv = idx_v[b], wt_v[b], pts_v[b]

            def vec_body(j, carry2):
                sb = j * LANES
                px = pv[pl.ds(sb, LANES)]
                py = pv[pl.ds(C + sb, LANES)]
                pz = pv[pl.ds(2 * C + sb, LANES)]
                x0, x1, xd = _axis_terms(px)
                y0, y1, yd = _axis_terms(py)
                z0, z1, zd = _axis_terms(pz)
                x0s = x0 << 16
                x1s = x1 << 16
                y0s = y0 << 8
                y1s = y1 << 8
                b00 = x0s + y0s
                b01 = x0s + y1s
                b10 = x1s + y0s
                b11 = x1s + y1s
                iv[pl.ds(0 * C + sb, LANES)] = b00 + z0
                iv[pl.ds(1 * C + sb, LANES)] = b00 + z1
                iv[pl.ds(2 * C + sb, LANES)] = b01 + z0
                iv[pl.ds(3 * C + sb, LANES)] = b01 + z1
                iv[pl.ds(4 * C + sb, LANES)] = b10 + z0
                iv[pl.ds(5 * C + sb, LANES)] = b10 + z1
                iv[pl.ds(6 * C + sb, LANES)] = b11 + z0
                iv[pl.ds(7 * C + sb, LANES)] = b11 + z1
                wv[pl.ds(0 * C + sb, LANES)] = xd
                wv[pl.ds(1 * C + sb, LANES)] = yd
                wv[pl.ds(2 * C + sb, LANES)] = zd
                return carry2

            lax.fori_loop(0, C // LANES, vec_body, 0)

        def combine_out(t, b):
            vv, wv = val_v[b], wt_v[b]
            for k in range(8):
                gather_copy(k, b).wait()

            def mix_body(j, carry2):
                sb = j * LANES
                v000 = vv[pl.ds(0 * C + sb, LANES)]
                v001 = vv[pl.ds(1 * C + sb, LANES)]
                v010 = vv[pl.ds(2 * C + sb, LANES)]
                v011 = vv[pl.ds(3 * C + sb, LANES)]
                v100 = vv[pl.ds(4 * C + sb, LANES)]
                v101 = vv[pl.ds(5 * C + sb, LANES)]
                v110 = vv[pl.ds(6 * C + sb, LANES)]
                v111 = vv[pl.ds(7 * C + sb, LANES)]
                xd = wv[pl.ds(0 * C + sb, LANES)]
                yd = wv[pl.ds(1 * C + sb, LANES)]
                zd = wv[pl.ds(2 * C + sb, LANES)]
                c00 = v000 + (v100 - v000) * xd
                c01 = v001 + (v101 - v001) * xd
                c10 = v010 + (v110 - v010) * xd
                c11 = v011 + (v111 - v011) * xd
                c0 = c00 + (c10 - c00) * yd
                c1 = c01 + (c11 - c01) * yd
                out_v[pl.ds(sb, LANES)] = c0 + (c1 - c0) * zd
                return carry2

            lax.fori_loop(0, C // LANES, mix_body, 0)
            pltpu.sync_copy(out_v, out_hbm.at[pl.ds(chunk_base(t), C)])

        def half_iter(t, cur):
            nxt = 1 - cur
            pts_wait(t, cur)
            compute_idx(cur)
            for k in range(8):
                gather_copy(k, cur).start()

            @pl.when(t + 1 < niter)
            def _():
                pts_start(t + 1, nxt)

            @pl.when(t >= 1)
            def _():
                combine_out(t - 1, nxt)

        pts_start(0, 0)

        def pair_body(tt, carry):
            half_iter(2 * tt, 0)
            half_iter(2 * tt + 1, 1)
            return carry

        lax.fori_loop(0, niter // 2, pair_body, 0)
        if niter % 2:
            half_iter(niter - 1, 0)
        last = niter - 1
        combine_out(last, last % 2)

    return sc_interp


def kernel(points, grid):
    npts = points.shape[0]
    niter = -(-npts // (C * NW))
    xs, ys, zs = points[:, 0], points[:, 1], points[:, 2]
    gridf = grid.reshape(-1)
    return _make_sc_interp(npts, niter)(xs, ys, zs, gridf)
